# baseline (speedup 1.0000x reference)
import jax
import jax.numpy as jnp
import numpy as np
from jax.experimental import pallas as pl
from jax.experimental.pallas import tpu as pltpu

_N = 22
_B = 32
_T = 64
_BT = _B * _T
_ROWS = _N * _BT
_GB = 128
_RB = _N * _GB
_NBLK = _ROWS // _RB
_H = 256
_G4 = 4 * _H
_BPB = _GB // _T


def _build_static():
    self_link = [(i, i) for i in range(_N)]
    neighbor_link = [(1, 0), (2, 1), (3, 2), (4, 3), (5, 0), (6, 5), (7, 6),
                     (8, 7), (9, 0), (10, 9), (11, 10), (12, 11), (13, 12),
                     (14, 11), (15, 14), (16, 15), (17, 16), (18, 11),
                     (19, 18), (20, 19), (21, 20)]
    edges = self_link + neighbor_link
    deg = np.ones(_N, dtype=np.float64)
    for _s, d in edges:
        deg[d] += float(_T)
    dinv = deg ** -0.5
    amat = np.zeros((_N, _N), dtype=np.float64)
    for s, d in edges:
        amat[d, s] += float(_T) * dinv[s] * dinv[d]
    amat += np.diag(dinv * dinv)
    pool = np.zeros((_GB, _RB), dtype=np.float64)
    for g in range(_GB):
        pool[g, g * _N:(g + 1) * _N] = 1.0
    return amat.astype(np.float32), pool.astype(np.float32)


_A22, _POOL = _build_static()


def _mm(a, b):
    return jax.lax.dot_general(a, b, (((1,), (0,)), ((), ())),
                               preferred_element_type=jnp.float32)


def _mmb(a, b):
    return jax.lax.dot_general(a.astype(jnp.bfloat16), b,
                               (((1,), (0,)), ((), ())),
                               preferred_element_type=jnp.float32)


def _mmbb(a, b):
    return jax.lax.dot_general(a.astype(jnp.bfloat16), b,
                               (((1,), (0,)), ((), ())),
                               preferred_element_type=jnp.float32
                               ).astype(jnp.bfloat16)


def _cell(gates, c_prev):
    gi = jax.nn.sigmoid(gates[:, 0:_H])
    gf = jax.nn.sigmoid(gates[:, _H:2 * _H])
    gg = jnp.tanh(gates[:, 2 * _H:3 * _H])
    go = jax.nn.sigmoid(gates[:, 3 * _H:4 * _H])
    c = gf * c_prev + gi * gg
    return go * jnp.tanh(c), c


def _fused_body(x_ref, w1_ref, b1_ref, w2_ref, b2_ref, w3_ref, b3_ref,
                w4_ref, b4_ref, a_ref, p_ref, wf_ref, bgf_ref, wb_ref,
                bgb_ref, whf_ref, whb_ref, wfc_ref, bfc_ref,
                out_ref, gxf_s, gxb_s):
    i = pl.program_id(0)

    @pl.when(i < _NBLK)
    def _phase_a():
        x = x_ref[...]
        h = x
        m = x[0:_N, :]
        amat = a_ref[...]
        for w_ref, b_ref in ((w1_ref, b1_ref), (w2_ref, b2_ref),
                             (w3_ref, b3_ref), (w4_ref, b4_ref)):
            w = w_ref[...]
            b = b_ref[...]
            h = jnp.maximum(_mmb(h, w) + b, 0.0)
            m = jnp.maximum(_mm(amat, _mmb(m, w)) + b, 0.0)
        y = _mmb(p_ref[...], h) * (1.0 / _N)
        ym = jnp.mean(m, axis=0, keepdims=True)
        rid = jax.lax.broadcasted_iota(jnp.int32, (_GB, 1), 0)
        y = jnp.where(jnp.logical_and(rid == 0, i == 0), ym, y)
        yb = y.astype(jnp.bfloat16)
        gf = _mm(yb, wf_ref[...]) + bgf_ref[...]
        gb = _mm(yb, wb_ref[...]) + bgb_ref[...]
        gxf_s[pl.ds(i * _BPB, _BPB)] = gf.reshape(_BPB, _T, _G4)
        gxb_s[pl.ds(i * _BPB, _BPB)] = gb.reshape(_BPB, _T, _G4)

    @pl.when(i == _NBLK)
    def _phase_b():
        whf = whf_ref[...]
        whb = whb_ref[...]

        def step(t, carry):
            hf, cf, hb, cb, hsf, hsb = carry
            xf = gxf_s[:, pl.ds(t, 1), :].reshape(_B, _G4)
            hf, cf = _cell(xf + _mmb(hf, whf), cf)
            xb = gxb_s[:, pl.ds(_T - 1 - t, 1), :].reshape(_B, _G4)
            hb, cb = _cell(xb + _mmb(hb, whb), cb)
            return hf, cf, hb, cb, hsf + hf, hsb + hb

        z = jnp.zeros((_B, _H), dtype=jnp.float32)
        hf, cf, hb, cb, hsf, hsb = jax.lax.fori_loop(
            0, _T, step, (z, z, z, z, z, z), unroll=8)
        wfc = wfc_ref[...]
        acc = _mm(hsf, wfc[0:_H, :]) + _mm(hsb, wfc[_H:2 * _H, :])
        out_ref[...] = acc * (1.0 / _T) + bfc_ref[...]


def kernel(data, W1, b1, W2, b2, W3, b3, W4, b4, Wih_f, Whh_f, bih_f, bhh_f,
           Wih_b, Whh_b, bih_b, bhh_b, Wfc, bfc):
    xflat = data.reshape(_BT, _N, 3).transpose(1, 0, 2).reshape(_ROWS, 3)
    bf = jnp.bfloat16
    const = lambda i: (0, 0)
    out = pl.pallas_call(
        _fused_body,
        grid=(_NBLK + 1,),
        in_specs=[
            pl.BlockSpec((_RB, 3), lambda i: (jnp.minimum(i, _NBLK - 1), 0)),
            pl.BlockSpec((3, 64), const), pl.BlockSpec((1, 64), const),
            pl.BlockSpec((64, 128), const), pl.BlockSpec((1, 128), const),
            pl.BlockSpec((128, 256), const), pl.BlockSpec((1, 256), const),
            pl.BlockSpec((256, 512), const), pl.BlockSpec((1, 512), const),
            pl.BlockSpec((_N, _N), const),
            pl.BlockSpec((_GB, _RB), const),
            pl.BlockSpec((512, _G4), const), pl.BlockSpec((1, _G4), const),
            pl.BlockSpec((512, _G4), const), pl.BlockSpec((1, _G4), const),
            pl.BlockSpec((_H, _G4), const),
            pl.BlockSpec((_H, _G4), const),
            pl.BlockSpec((2 * _H, 2 * _H), const),
            pl.BlockSpec((1, 2 * _H), const),
        ],
        out_specs=pl.BlockSpec((_B, 2 * _H), const),
        out_shape=jax.ShapeDtypeStruct((_B, 2 * _H), jnp.float32),
        scratch_shapes=[pltpu.VMEM((_B, _T, _G4), jnp.float32),
                        pltpu.VMEM((_B, _T, _G4), jnp.float32)],
    )(xflat, W1.astype(bf), b1.reshape(1, -1).astype(bf), W2.astype(bf),
      b2.reshape(1, -1).astype(bf), W3.astype(bf), b3.reshape(1, -1).astype(bf),
      W4.astype(bf), b4.reshape(1, -1).astype(bf),
      jnp.asarray(_A22), jnp.asarray(_POOL).astype(bf),
      Wih_f.T.astype(bf), (bih_f + bhh_f).reshape(1, -1),
      Wih_b.T.astype(bf), (bih_b + bhh_b).reshape(1, -1),
      Whh_f.T.astype(bf), Whh_b.T.astype(bf), Wfc, bfc.reshape(1, -1))
    return out


# speedup vs baseline: 1.2110x; 1.2110x over previous
import jax
import jax.numpy as jnp
import numpy as np
from jax.experimental import pallas as pl
from jax.experimental.pallas import tpu as pltpu

_N = 22
_B = 32
_T = 64
_BT = _B * _T
_ROWS = _N * _BT
_GB = 256
_RB = _N * _GB
_NBLK = _ROWS // _RB
_H = 256
_G4 = 4 * _H
_BPB = _GB // _T


def _build_static():
    self_link = [(i, i) for i in range(_N)]
    neighbor_link = [(1, 0), (2, 1), (3, 2), (4, 3), (5, 0), (6, 5), (7, 6),
                     (8, 7), (9, 0), (10, 9), (11, 10), (12, 11), (13, 12),
                     (14, 11), (15, 14), (16, 15), (17, 16), (18, 11),
                     (19, 18), (20, 19), (21, 20)]
    edges = self_link + neighbor_link
    deg = np.ones(_N, dtype=np.float64)
    for _s, d in edges:
        deg[d] += float(_T)
    dinv = deg ** -0.5
    amat = np.zeros((_N, _N), dtype=np.float64)
    for s, d in edges:
        amat[d, s] += float(_T) * dinv[s] * dinv[d]
    amat += np.diag(dinv * dinv)
    pool = np.zeros((_T, _N * _T), dtype=np.float64)
    for g in range(_T):
        pool[g, g * _N:(g + 1) * _N] = 1.0
    return amat.astype(np.float32), pool.astype(np.float32)


_A22, _POOL = _build_static()


def _mm(a, b):
    return jax.lax.dot_general(a, b, (((1,), (0,)), ((), ())),
                               preferred_element_type=jnp.float32)


def _mmb(a, b):
    return jax.lax.dot_general(a.astype(jnp.bfloat16), b,
                               (((1,), (0,)), ((), ())),
                               preferred_element_type=jnp.float32)


def _mmbb(a, b):
    return jax.lax.dot_general(a.astype(jnp.bfloat16), b,
                               (((1,), (0,)), ((), ())),
                               preferred_element_type=jnp.float32
                               ).astype(jnp.bfloat16)


def _cell(gates, c_prev):
    gi = jax.nn.sigmoid(gates[:, 0:_H])
    gf = jax.nn.sigmoid(gates[:, _H:2 * _H])
    gg = jnp.tanh(gates[:, 2 * _H:3 * _H])
    go = jax.nn.sigmoid(gates[:, 3 * _H:4 * _H])
    c = gf * c_prev + gi * gg
    return go * jnp.tanh(c), c


def _fused_body(x_ref, w1_ref, b1_ref, w2_ref, b2_ref, w3_ref, b3_ref,
                w4_ref, b4_ref, a_ref, p_ref, wf_ref, bgf_ref, wb_ref,
                bgb_ref, whf_ref, whb_ref, wfc_ref, bfc_ref,
                out_ref, gxf_s, gxb_s):
    i = pl.program_id(0)

    @pl.when(i < _NBLK)
    def _phase_a():
        x = x_ref[...]
        h = x
        m = x[0:_N, :]
        amat = a_ref[...]
        for w_ref, b_ref in ((w1_ref, b1_ref), (w2_ref, b2_ref),
                             (w3_ref, b3_ref), (w4_ref, b4_ref)):
            w = w_ref[...]
            b = b_ref[...]
            h = jnp.maximum(_mmb(h, w) + b, 0.0)
            m = jnp.maximum(_mm(amat, _mmb(m, w)) + b, 0.0)
        pb = p_ref[...]
        y = jnp.concatenate(
            [_mmb(pb, h[k * _N * _T:(k + 1) * _N * _T]) for k in range(_BPB)],
            axis=0) * (1.0 / _N)
        ym = jnp.mean(m, axis=0, keepdims=True)
        rid = jax.lax.broadcasted_iota(jnp.int32, (_GB, 1), 0)
        y = jnp.where(jnp.logical_and(rid == 0, i == 0), ym, y)
        yb = y.astype(jnp.bfloat16)
        gf = _mm(yb, wf_ref[...]) + bgf_ref[...]
        gb = _mm(yb, wb_ref[...]) + bgb_ref[...]
        gxf_s[pl.ds(i * _BPB, _BPB)] = gf.reshape(_BPB, _T, _G4)
        gxb_s[pl.ds(i * _BPB, _BPB)] = gb.reshape(_BPB, _T, _G4)

    @pl.when(i == _NBLK)
    def _phase_b():
        whf = whf_ref[...]
        whb = whb_ref[...]

        def step(t, carry):
            hf, cf, hb, cb, hsf, hsb = carry
            xf = gxf_s[:, pl.ds(t, 1), :].reshape(_B, _G4)
            hf, cf = _cell(xf + _mmb(hf, whf), cf)
            xb = gxb_s[:, pl.ds(_T - 1 - t, 1), :].reshape(_B, _G4)
            hb, cb = _cell(xb + _mmb(hb, whb), cb)
            return hf, cf, hb, cb, hsf + hf, hsb + hb

        z = jnp.zeros((_B, _H), dtype=jnp.float32)
        hf, cf, hb, cb, hsf, hsb = jax.lax.fori_loop(
            0, _T, step, (z, z, z, z, z, z), unroll=8)
        wfc = wfc_ref[...]
        acc = _mm(hsf, wfc[0:_H, :]) + _mm(hsb, wfc[_H:2 * _H, :])
        out_ref[...] = acc * (1.0 / _T) + bfc_ref[...]


def kernel(data, W1, b1, W2, b2, W3, b3, W4, b4, Wih_f, Whh_f, bih_f, bhh_f,
           Wih_b, Whh_b, bih_b, bhh_b, Wfc, bfc):
    xflat = data.reshape(_BT, _N, 3).transpose(1, 0, 2).reshape(_ROWS, 3)
    bf = jnp.bfloat16
    const = lambda i: (0, 0)
    out = pl.pallas_call(
        _fused_body,
        grid=(_NBLK + 1,),
        in_specs=[
            pl.BlockSpec((_RB, 3), lambda i: (jnp.minimum(i, _NBLK - 1), 0)),
            pl.BlockSpec((3, 64), const), pl.BlockSpec((1, 64), const),
            pl.BlockSpec((64, 128), const), pl.BlockSpec((1, 128), const),
            pl.BlockSpec((128, 256), const), pl.BlockSpec((1, 256), const),
            pl.BlockSpec((256, 512), const), pl.BlockSpec((1, 512), const),
            pl.BlockSpec((_N, _N), const),
            pl.BlockSpec((_T, _N * _T), const),
            pl.BlockSpec((512, _G4), const), pl.BlockSpec((1, _G4), const),
            pl.BlockSpec((512, _G4), const), pl.BlockSpec((1, _G4), const),
            pl.BlockSpec((_H, _G4), const),
            pl.BlockSpec((_H, _G4), const),
            pl.BlockSpec((2 * _H, 2 * _H), const),
            pl.BlockSpec((1, 2 * _H), const),
        ],
        out_specs=pl.BlockSpec((_B, 2 * _H), const),
        out_shape=jax.ShapeDtypeStruct((_B, 2 * _H), jnp.float32),
        scratch_shapes=[pltpu.VMEM((_B, _T, _G4), jnp.float32),
                        pltpu.VMEM((_B, _T, _G4), jnp.float32)],
    )(xflat, W1.astype(bf), b1.reshape(1, -1).astype(bf), W2.astype(bf),
      b2.reshape(1, -1).astype(bf), W3.astype(bf), b3.reshape(1, -1).astype(bf),
      W4.astype(bf), b4.reshape(1, -1).astype(bf),
      jnp.asarray(_A22), jnp.asarray(_POOL).astype(bf),
      Wih_f.T.astype(bf), (bih_f + bhh_f).reshape(1, -1),
      Wih_b.T.astype(bf), (bih_b + bhh_b).reshape(1, -1),
      Whh_f.T.astype(bf), Whh_b.T.astype(bf), Wfc, bfc.reshape(1, -1))
    return out
